# Initial kernel scaffold; baseline (speedup 1.0000x reference)
import jax
import jax.numpy as jnp
from jax.experimental import pallas as pl


def kernel(x, Wg, W1, W2, W3, sw1, sw2, sw3):
    raise NotImplementedError("write your pallas kernel here")


# speedup vs baseline: 1.0108x; 1.0108x over previous
import functools

import jax
import jax.numpy as jnp
from jax.experimental import pallas as pl
from jax.experimental.pallas import tpu as pltpu

E = 8
TOPK = 2
DIM = 1024
HID = 2752
N = 2048
NE = E + 1
BM = 256


def _gate_body(xf_ref, wg_ref, cw_ref):
    xf = xf_ref[...]
    logits = jax.lax.dot_general(
        xf, wg_ref[...], (((1,), (1,)), ((), ())),
        preferred_element_type=jnp.float32)
    m = jnp.max(logits, axis=1, keepdims=True)
    ex = jnp.exp(logits - m)
    s = ex / jnp.sum(ex, axis=1, keepdims=True)

    iota = jax.lax.broadcasted_iota(jnp.int32, (N, E), 1)
    m1 = jnp.max(s, axis=1, keepdims=True)
    i1 = jnp.min(jnp.where(s == m1, iota, E), axis=1, keepdims=True)
    top1 = iota == i1
    s2 = jnp.where(top1, -jnp.inf, s)
    m2 = jnp.max(s2, axis=1, keepdims=True)
    i2 = jnp.min(jnp.where(s2 == m2, iota, E), axis=1, keepdims=True)

    iota16 = jax.lax.broadcasted_iota(jnp.int32, (N, 16), 1)
    cw = (jnp.where(iota16 == i1, m1, 0.0)
          + jnp.where(iota16 == i2, m2, 0.0))
    cw_ref[...] = jnp.where(iota16 == E, 1.0, cw)


def _expert_body(cw_ref, xf_ref, w1_ref, w3_ref, w2_ref, out_ref):
    e = pl.program_id(0)
    t = pl.program_id(1)
    rows = pl.ds(t * BM, BM)
    xb = xf_ref[rows, :].astype(jnp.bfloat16)
    h1 = jax.lax.dot_general(xb, w1_ref[...], (((1,), (1,)), ((), ())),
                             preferred_element_type=jnp.float32)
    h3 = jax.lax.dot_general(xb, w3_ref[...], (((1,), (1,)), ((), ())),
                             preferred_element_type=jnp.float32)
    h = (h1 * jax.lax.logistic(h1) * h3).astype(jnp.bfloat16)
    y = jax.lax.dot_general(h, w2_ref[...], (((1,), (1,)), ((), ())),
                            preferred_element_type=jnp.float32)
    iota16 = jax.lax.broadcasted_iota(jnp.int32, (BM, 16), 1)
    scale = jnp.sum(jnp.where(iota16 == e, cw_ref[rows, :], 0.0),
                    axis=1, keepdims=True)
    ys = scale * y

    @pl.when(e == 0)
    def _init():
        out_ref[rows, :] = ys

    @pl.when(e != 0)
    def _acc():
        out_ref[rows, :] += ys


def kernel(x, Wg, W1, W2, W3, sw1, sw2, sw3):
    xf = x.reshape(N, DIM)
    w1a = jnp.concatenate([W1, sw1[None]], axis=0).astype(jnp.bfloat16)
    w3a = jnp.concatenate([W3, sw3[None]], axis=0).astype(jnp.bfloat16)
    w2a = jnp.concatenate([W2, sw2[None]], axis=0).astype(jnp.bfloat16)

    cw = pl.pallas_call(
        _gate_body,
        out_shape=jax.ShapeDtypeStruct((N, 16), jnp.float32),
        in_specs=[pl.BlockSpec((N, DIM), lambda: (0, 0)),
                  pl.BlockSpec((E, DIM), lambda: (0, 0))],
        out_specs=pl.BlockSpec((N, 16), lambda: (0, 0)),
    )(xf, Wg)

    y = pl.pallas_call(
        _expert_body,
        grid=(NE, N // BM),
        out_shape=jax.ShapeDtypeStruct((N, DIM), jnp.float32),
        in_specs=[
            pl.BlockSpec((N, 16), lambda e, t: (0, 0)),
            pl.BlockSpec((N, DIM), lambda e, t: (0, 0)),
            pl.BlockSpec((None, HID, DIM), lambda e, t: (e, 0, 0)),
            pl.BlockSpec((None, HID, DIM), lambda e, t: (e, 0, 0)),
            pl.BlockSpec((None, DIM, HID), lambda e, t: (e, 0, 0)),
        ],
        out_specs=pl.BlockSpec((N, DIM), lambda e, t: (0, 0)),
        compiler_params=pltpu.CompilerParams(
            dimension_semantics=("arbitrary", "arbitrary")),
    )(cw, xf, w1a, w3a, w2a)

    return y.reshape(x.shape).astype(x.dtype)


# speedup vs baseline: 1.3266x; 1.3124x over previous
import functools

import jax
import jax.numpy as jnp
from jax import lax
from jax.experimental import pallas as pl
from jax.experimental.pallas import tpu as pltpu
from jax.experimental.pallas import tpu_sc as plsc

E = 8
DIM = 1024
HID = 2752
N = 2048
BM = 256
NBLK_ROUTED = 24
NBLK_SHARED = N // BM
TOT_BLKS = NBLK_ROUTED + NBLK_SHARED
SHOFF = NBLK_ROUTED * BM
CAP = TOT_BLKS * BM

NW = 32
TPW = N // NW
CCH = 16


def _gate_body(xf_ref, wg_ref, pos_ref, w_ref, gid_ref, totr_ref):
    xf = xf_ref[...]
    logits = lax.dot_general(xf, wg_ref[...], (((1,), (1,)), ((), ())),
                             preferred_element_type=jnp.float32)
    m = jnp.max(logits, axis=1, keepdims=True)
    ex = jnp.exp(logits - m)
    s = ex / jnp.sum(ex, axis=1, keepdims=True)

    iota = lax.broadcasted_iota(jnp.int32, (N, E), 1)
    m1 = jnp.max(s, axis=1, keepdims=True)
    i1 = jnp.min(jnp.where(s == m1, iota, E), axis=1, keepdims=True)
    top1 = iota == i1
    s2 = jnp.where(top1, -jnp.inf, s)
    m2 = jnp.max(s2, axis=1, keepdims=True)
    i2 = jnp.min(jnp.where(s2 == m2, iota, E), axis=1, keepdims=True)
    h1 = top1.astype(jnp.float32)
    h2 = (iota == i2).astype(jnp.float32)

    nchunk = N // 128
    ci = lax.broadcasted_iota(jnp.int32, (128, 128), 0)
    cj = lax.broadcasted_iota(jnp.int32, (128, 128), 1)
    tril = (ci >= cj).astype(jnp.float32)

    def ranks(h):
        run = jnp.zeros((1, E), jnp.float32)
        outs = []
        for b in range(nchunk):
            hb = h[b * 128:(b + 1) * 128, :]
            incl = lax.dot_general(tril, hb, (((1,), (0,)), ((), ())),
                                   preferred_element_type=jnp.float32)
            outs.append(incl - hb + run)
            run = run + incl[127:128, :]
        return jnp.concatenate(outs, axis=0), run

    r1, tot1 = ranks(h1)
    r2, tot2 = ranks(h2)
    cnt = tot1 + tot2
    nb = jnp.ceil(cnt * (1.0 / BM))
    ei = lax.broadcasted_iota(jnp.int32, (E, E), 0)
    ej = lax.broadcasted_iota(jnp.int32, (E, E), 1)
    sl = (ei < ej).astype(jnp.float32)
    boff = lax.dot_general(nb, sl, (((1,), (0,)), ((), ())),
                           preferred_element_type=jnp.float32)
    cumnext = boff + nb
    totr = cumnext[0:1, E - 1:E]

    base = boff * BM
    pos1 = jnp.sum(h1 * (base + r1), axis=1, keepdims=True)
    pos2 = jnp.sum(h2 * (base + tot1 + r2), axis=1, keepdims=True)
    pos_ref[...] = jnp.concatenate([pos1, pos2], axis=1).astype(jnp.int32)
    w_ref[...] = jnp.concatenate([m1, m2], axis=1)

    bi = lax.broadcasted_iota(jnp.int32, (TOT_BLKS, E), 0)
    gid_ref[...] = jnp.sum((bi >= cumnext.astype(jnp.int32)).astype(jnp.int32),
                           axis=1, keepdims=True)
    totr_ref[...] = totr.astype(jnp.int32)


def _gmm_body(gid_ref, totr_ref, posf_ref, xb_ref, w1_ref, w3_ref,
              w2_ref, ys_ref):
    b = pl.program_id(0)
    routed = b < NBLK_ROUTED

    @pl.when(jnp.logical_or(b < totr_ref[0], b >= NBLK_ROUTED))
    def _():
        gp = lax.broadcasted_iota(jnp.int32, (BM, N), 0) + b * BM
        p0 = posf_ref[0:1, :]
        p1 = posf_ref[1:2, :]
        oh = jnp.logical_or(gp == p0, gp == p1).astype(jnp.bfloat16)
        xg = lax.dot_general(oh, xb_ref[...], (((1,), (0,)), ((), ())),
                             preferred_element_type=jnp.float32
                             ).astype(jnp.bfloat16)
        xsh = xb_ref[pl.ds(jnp.maximum(b - NBLK_ROUTED, 0) * BM, BM), :]
        xb = jnp.where(routed, xg, xsh)
        h1 = lax.dot_general(xb, w1_ref[...], (((1,), (1,)), ((), ())),
                             preferred_element_type=jnp.float32)
        h3 = lax.dot_general(xb, w3_ref[...], (((1,), (1,)), ((), ())),
                             preferred_element_type=jnp.float32)
        h = (h1 * lax.logistic(h1) * h3).astype(jnp.bfloat16)
        ys_ref[...] = lax.dot_general(h, w2_ref[...], (((1,), (1,)), ((), ())),
                                      preferred_element_type=jnp.float32)


def _sc_combine_body(ys_hbm, posf_hbm, wf_hbm, y_hbm,
                     idx0_v, idx1_v, w0_v, w1_v, g0_v, g1_v, sh_v, out_v,
                     sem_g, sem_o):
    wid = lax.axis_index("s") * 2 + lax.axis_index("c")
    base = wid * TPW
    nch = TPW // CCH
    for c in range(nch):
        cb = base + c * CCH
        i0 = pltpu.async_copy(posf_hbm.at[pl.ds(cb, CCH)], idx0_v, sem_g)
        i1 = pltpu.async_copy(posf_hbm.at[pl.ds(N + cb, CCH)], idx1_v, sem_g)
        i2 = pltpu.async_copy(wf_hbm.at[pl.ds(cb, CCH)], w0_v, sem_g)
        i3 = pltpu.async_copy(wf_hbm.at[pl.ds(N + cb, CCH)], w1_v, sem_g)
        i0.wait(); i1.wait(); i2.wait(); i3.wait()
        g0 = pltpu.async_copy(ys_hbm.at[idx0_v], g0_v, sem_g)
        g1 = pltpu.async_copy(ys_hbm.at[idx1_v], g1_v, sem_g)
        gs = pltpu.async_copy(ys_hbm.at[pl.ds(SHOFF + cb, CCH)], sh_v, sem_g)
        g0.wait(); g1.wait(); gs.wait()
        w0c = w0_v[...]
        w1c = w1_v[...]

        dnums = lax.GatherDimensionNumbers(
            offset_dims=(), collapsed_slice_dims=(0,), start_index_map=(0,))

        def row(r, _):
            ri = jnp.full((CCH, 1), r, jnp.int32)
            w0r = lax.gather(w0c, ri, dnums, (1,),
                             mode=lax.GatherScatterMode.PROMISE_IN_BOUNDS)
            w1r = lax.gather(w1c, ri, dnums, (1,),
                             mode=lax.GatherScatterMode.PROMISE_IN_BOUNDS)
            for v in range(DIM // 16):
                s = pl.ds(v * 16, 16)
                out_v[r, s] = (w0r * g0_v[r, s] + w1r * g1_v[r, s]
                               + sh_v[r, s])
            return 0

        lax.fori_loop(0, CCH, row, 0)
        ow = pltpu.async_copy(out_v, y_hbm.at[pl.ds(cb, CCH)], sem_o)
        ow.wait()


_SC_MESH = plsc.VectorSubcoreMesh(core_axis_name="c", subcore_axis_name="s",
                                  num_cores=2, num_subcores=16)

_sc_combine = functools.partial(
    pl.kernel,
    out_type=jax.ShapeDtypeStruct((N, DIM), jnp.float32),
    mesh=_SC_MESH,
    scratch_types=[
        pltpu.VMEM((CCH,), jnp.int32),
        pltpu.VMEM((CCH,), jnp.int32),
        pltpu.VMEM((CCH,), jnp.float32),
        pltpu.VMEM((CCH,), jnp.float32),
        pltpu.VMEM((CCH, DIM), jnp.float32),
        pltpu.VMEM((CCH, DIM), jnp.float32),
        pltpu.VMEM((CCH, DIM), jnp.float32),
        pltpu.VMEM((CCH, DIM), jnp.float32),
        pltpu.SemaphoreType.DMA,
        pltpu.SemaphoreType.DMA,
    ],
)(_sc_combine_body)


def kernel(x, Wg, W1, W2, W3, sw1, sw2, sw3):
    xf = x.reshape(N, DIM)
    w1a = jnp.concatenate([W1, sw1[None]], axis=0).astype(jnp.bfloat16)
    w3a = jnp.concatenate([W3, sw3[None]], axis=0).astype(jnp.bfloat16)
    w2a = jnp.concatenate([W2, sw2[None]], axis=0).astype(jnp.bfloat16)

    pos2, w2, gid, totr = pl.pallas_call(
        _gate_body,
        out_shape=[
            jax.ShapeDtypeStruct((N, 2), jnp.int32),
            jax.ShapeDtypeStruct((N, 2), jnp.float32),
            jax.ShapeDtypeStruct((TOT_BLKS, 1), jnp.int32),
            jax.ShapeDtypeStruct((1, 1), jnp.int32),
        ],
        in_specs=[pl.BlockSpec((N, DIM), lambda: (0, 0)),
                  pl.BlockSpec((E, DIM), lambda: (0, 0))],
        out_specs=[pl.BlockSpec((N, 2), lambda: (0, 0)),
                   pl.BlockSpec((N, 2), lambda: (0, 0)),
                   pl.BlockSpec((TOT_BLKS, 1), lambda: (0, 0)),
                   pl.BlockSpec((1, 1), lambda: (0, 0))],
    )(xf, Wg)

    posr = pos2.T.reshape(2, N)
    posf = posr.reshape(2 * N)
    wf = w2.T.reshape(2 * N)
    xb16 = xf.astype(jnp.bfloat16)

    ys = pl.pallas_call(
        _gmm_body,
        grid_spec=pltpu.PrefetchScalarGridSpec(
            num_scalar_prefetch=2,
            grid=(TOT_BLKS,),
            in_specs=[
                pl.BlockSpec((2, N), lambda b, gid, totr: (0, 0)),
                pl.BlockSpec((N, DIM), lambda b, gid, totr: (0, 0)),
                pl.BlockSpec((None, HID, DIM),
                             lambda b, gid, totr: (gid[b], 0, 0)),
                pl.BlockSpec((None, HID, DIM),
                             lambda b, gid, totr: (gid[b], 0, 0)),
                pl.BlockSpec((None, DIM, HID),
                             lambda b, gid, totr: (gid[b], 0, 0)),
            ],
            out_specs=pl.BlockSpec((BM, DIM), lambda b, gid, totr: (b, 0)),
        ),
        out_shape=jax.ShapeDtypeStruct((CAP, DIM), jnp.float32),
        compiler_params=pltpu.CompilerParams(
            dimension_semantics=("arbitrary",)),
    )(gid.reshape(TOT_BLKS), totr.reshape(1), posr, xb16, w1a, w3a, w2a)

    y = _sc_combine(ys, posf, wf)
    return y.reshape(x.shape).astype(x.dtype)


# speedup vs baseline: 1.3408x; 1.0107x over previous
import functools

import jax
import jax.numpy as jnp
from jax import lax
from jax.experimental import pallas as pl
from jax.experimental.pallas import tpu as pltpu
from jax.experimental.pallas import tpu_sc as plsc

E = 8
DIM = 1024
HID = 2752
N = 2048
BM = 256
NBLK_ROUTED = 24
NBLK_SHARED = N // BM
SHROWS = NBLK_ROUTED * BM
NHK = 4
HB = HID // NHK

NW = 32
TPW = N // NW
CCH = 16


def _gate_body(xf_ref, wg_ref, pos_ref, w_ref, gid_ref, totr_ref):
    xf = xf_ref[...]
    logits = lax.dot_general(xf, wg_ref[...], (((1,), (1,)), ((), ())),
                             preferred_element_type=jnp.float32)
    m = jnp.max(logits, axis=1, keepdims=True)
    ex = jnp.exp(logits - m)
    s = ex / jnp.sum(ex, axis=1, keepdims=True)

    iota = lax.broadcasted_iota(jnp.int32, (N, E), 1)
    m1 = jnp.max(s, axis=1, keepdims=True)
    i1 = jnp.min(jnp.where(s == m1, iota, E), axis=1, keepdims=True)
    top1 = iota == i1
    s2 = jnp.where(top1, -jnp.inf, s)
    m2 = jnp.max(s2, axis=1, keepdims=True)
    i2 = jnp.min(jnp.where(s2 == m2, iota, E), axis=1, keepdims=True)
    h1 = top1.astype(jnp.float32)
    h2 = (iota == i2).astype(jnp.float32)

    nchunk = N // 128
    ci = lax.broadcasted_iota(jnp.int32, (128, 128), 0)
    cj = lax.broadcasted_iota(jnp.int32, (128, 128), 1)
    tril = (ci >= cj).astype(jnp.float32)

    def ranks(h):
        run = jnp.zeros((1, E), jnp.float32)
        outs = []
        for b in range(nchunk):
            hb = h[b * 128:(b + 1) * 128, :]
            incl = lax.dot_general(tril, hb, (((1,), (0,)), ((), ())),
                                   preferred_element_type=jnp.float32)
            outs.append(incl - hb + run)
            run = run + incl[127:128, :]
        return jnp.concatenate(outs, axis=0), run

    r1, tot1 = ranks(h1)
    r2, tot2 = ranks(h2)
    cnt = tot1 + tot2
    nb = jnp.ceil(cnt * (1.0 / BM))
    ei = lax.broadcasted_iota(jnp.int32, (E, E), 0)
    ej = lax.broadcasted_iota(jnp.int32, (E, E), 1)
    sl = (ei < ej).astype(jnp.float32)
    boff = lax.dot_general(nb, sl, (((1,), (0,)), ((), ())),
                           preferred_element_type=jnp.float32)
    cumnext = boff + nb
    totr = cumnext[0:1, E - 1:E]

    base = boff * BM
    pos1 = jnp.sum(h1 * (base + r1), axis=1, keepdims=True)
    pos2 = jnp.sum(h2 * (base + tot1 + r2), axis=1, keepdims=True)
    pos_ref[...] = jnp.concatenate([pos1, pos2], axis=1).astype(jnp.int32)
    w_ref[...] = jnp.concatenate([m1, m2], axis=1)

    bi = lax.broadcasted_iota(jnp.int32, (NBLK_ROUTED, E), 0)
    gid_ref[...] = jnp.sum((bi >= cumnext.astype(jnp.int32)).astype(jnp.int32),
                           axis=1, keepdims=True)
    totr_ref[...] = totr.astype(jnp.int32)


def _swiglu_steps(hk, xg, w1_ref, w3_ref, w2_ref, h_ref, out_ref):
    h1 = lax.dot_general(xg, w1_ref[...].astype(jnp.bfloat16),
                         (((1,), (1,)), ((), ())),
                         preferred_element_type=jnp.float32)
    h3 = lax.dot_general(xg, w3_ref[...].astype(jnp.bfloat16),
                         (((1,), (1,)), ((), ())),
                         preferred_element_type=jnp.float32)
    hp = (h1 * lax.logistic(h1) * h3).astype(jnp.bfloat16)
    for k in range(NHK):
        @pl.when(hk == k)
        def _store():
            h_ref[:, k * HB:(k + 1) * HB] = hp

    @pl.when(hk == NHK - 1)
    def _out():
        out_ref[...] = lax.dot_general(
            h_ref[...], w2_ref[...].astype(jnp.bfloat16),
            (((1,), (1,)), ((), ())), preferred_element_type=jnp.float32)


def _gmm_body(gid_ref, totr_ref, posf_ref, xb_ref, w1_ref, w3_ref,
              w2_ref, ys_ref, xg_ref, h_ref):
    b = pl.program_id(0)
    hk = pl.program_id(1)

    @pl.when(b < totr_ref[0])
    def _():
        @pl.when(hk == 0)
        def _gather():
            gp = lax.broadcasted_iota(jnp.int32, (BM, N), 0) + b * BM
            oh = jnp.logical_or(gp == posf_ref[0:1, :],
                                gp == posf_ref[1:2, :]).astype(jnp.bfloat16)
            xg_ref[...] = lax.dot_general(
                oh, xb_ref[...], (((1,), (0,)), ((), ())),
                preferred_element_type=jnp.float32).astype(jnp.bfloat16)

        _swiglu_steps(hk, xg_ref[...], w1_ref, w3_ref, w2_ref, h_ref, ys_ref)


def _shared_body(xb_ref, w1_ref, w3_ref, w2_ref, ysh_ref, h_ref):
    hk = pl.program_id(1)
    _swiglu_steps(hk, xb_ref[...], w1_ref, w3_ref, w2_ref, h_ref, ysh_ref)


def _sc_combine_body(ys_hbm, ysh_hbm, posf_hbm, wf_hbm, y_hbm,
                     idx0_v, idx1_v, w0_v, w1_v, g0_v, g1_v, sh_v, out_v,
                     sem_g, sem_o):
    wid = lax.axis_index("s") * 2 + lax.axis_index("c")
    base = wid * TPW
    nch = TPW // CCH
    for c in range(nch):
        cb = base + c * CCH
        i0 = pltpu.async_copy(posf_hbm.at[pl.ds(cb, CCH)], idx0_v, sem_g)
        i1 = pltpu.async_copy(posf_hbm.at[pl.ds(N + cb, CCH)], idx1_v, sem_g)
        i2 = pltpu.async_copy(wf_hbm.at[pl.ds(cb, CCH)], w0_v, sem_g)
        i3 = pltpu.async_copy(wf_hbm.at[pl.ds(N + cb, CCH)], w1_v, sem_g)
        i0.wait(); i1.wait(); i2.wait(); i3.wait()
        g0 = pltpu.async_copy(ys_hbm.at[idx0_v], g0_v, sem_g)
        g1 = pltpu.async_copy(ys_hbm.at[idx1_v], g1_v, sem_g)
        gs = pltpu.async_copy(ysh_hbm.at[pl.ds(cb, CCH)], sh_v, sem_g)
        g0.wait(); g1.wait(); gs.wait()
        w0c = w0_v[...]
        w1c = w1_v[...]

        dnums = lax.GatherDimensionNumbers(
            offset_dims=(), collapsed_slice_dims=(0,), start_index_map=(0,))

        def row(r, _):
            ri = jnp.full((CCH, 1), r, jnp.int32)
            w0r = lax.gather(w0c, ri, dnums, (1,),
                             mode=lax.GatherScatterMode.PROMISE_IN_BOUNDS)
            w1r = lax.gather(w1c, ri, dnums, (1,),
                             mode=lax.GatherScatterMode.PROMISE_IN_BOUNDS)
            for v in range(DIM // 16):
                s = pl.ds(v * 16, 16)
                out_v[r, s] = (w0r * g0_v[r, s] + w1r * g1_v[r, s]
                               + sh_v[r, s])
            return 0

        lax.fori_loop(0, CCH, row, 0)
        ow = pltpu.async_copy(out_v, y_hbm.at[pl.ds(cb, CCH)], sem_o)
        ow.wait()


_SC_MESH = plsc.VectorSubcoreMesh(core_axis_name="c", subcore_axis_name="s",
                                  num_cores=2, num_subcores=16)

_sc_combine = functools.partial(
    pl.kernel,
    out_type=jax.ShapeDtypeStruct((N, DIM), jnp.float32),
    mesh=_SC_MESH,
    scratch_types=[
        pltpu.VMEM((CCH,), jnp.int32),
        pltpu.VMEM((CCH,), jnp.int32),
        pltpu.VMEM((CCH,), jnp.float32),
        pltpu.VMEM((CCH,), jnp.float32),
        pltpu.VMEM((CCH, DIM), jnp.float32),
        pltpu.VMEM((CCH, DIM), jnp.float32),
        pltpu.VMEM((CCH, DIM), jnp.float32),
        pltpu.VMEM((CCH, DIM), jnp.float32),
        pltpu.SemaphoreType.DMA,
        pltpu.SemaphoreType.DMA,
    ],
)(_sc_combine_body)


def kernel(x, Wg, W1, W2, W3, sw1, sw2, sw3):
    xf = x.reshape(N, DIM)

    pos2, w2, gid, totr = pl.pallas_call(
        _gate_body,
        out_shape=[
            jax.ShapeDtypeStruct((N, 2), jnp.int32),
            jax.ShapeDtypeStruct((N, 2), jnp.float32),
            jax.ShapeDtypeStruct((NBLK_ROUTED, 1), jnp.int32),
            jax.ShapeDtypeStruct((1, 1), jnp.int32),
        ],
        in_specs=[pl.BlockSpec((N, DIM), lambda: (0, 0)),
                  pl.BlockSpec((E, DIM), lambda: (0, 0))],
        out_specs=[pl.BlockSpec((N, 2), lambda: (0, 0)),
                   pl.BlockSpec((N, 2), lambda: (0, 0)),
                   pl.BlockSpec((NBLK_ROUTED, 1), lambda: (0, 0)),
                   pl.BlockSpec((1, 1), lambda: (0, 0))],
    )(xf, Wg)

    posr = pos2.T.reshape(2, N)
    posf = posr.reshape(2 * N)
    wf = w2.T.reshape(2 * N)
    xb16 = xf.astype(jnp.bfloat16)

    ys = pl.pallas_call(
        _gmm_body,
        grid_spec=pltpu.PrefetchScalarGridSpec(
            num_scalar_prefetch=2,
            grid=(NBLK_ROUTED, NHK),
            in_specs=[
                pl.BlockSpec((2, N), lambda b, hk, gid, totr: (0, 0)),
                pl.BlockSpec((N, DIM), lambda b, hk, gid, totr: (0, 0)),
                pl.BlockSpec((None, HB, DIM),
                             lambda b, hk, gid, totr: (gid[b], hk, 0)),
                pl.BlockSpec((None, HB, DIM),
                             lambda b, hk, gid, totr: (gid[b], hk, 0)),
                pl.BlockSpec((None, DIM, HID),
                             lambda b, hk, gid, totr: (gid[b], 0, 0)),
            ],
            out_specs=pl.BlockSpec((BM, DIM), lambda b, hk, gid, totr: (b, 0)),
            scratch_shapes=[pltpu.VMEM((BM, DIM), jnp.bfloat16),
                            pltpu.VMEM((BM, HID), jnp.bfloat16)],
        ),
        out_shape=jax.ShapeDtypeStruct((SHROWS, DIM), jnp.float32),
        compiler_params=pltpu.CompilerParams(
            dimension_semantics=("arbitrary", "arbitrary")),
    )(gid.reshape(NBLK_ROUTED), totr.reshape(1), posr, xb16, W1, W3, W2)

    ysh = pl.pallas_call(
        _shared_body,
        grid=(N // BM, NHK),
        in_specs=[
            pl.BlockSpec((BM, DIM), lambda t, hk: (t, 0)),
            pl.BlockSpec((HB, DIM), lambda t, hk: (hk, 0)),
            pl.BlockSpec((HB, DIM), lambda t, hk: (hk, 0)),
            pl.BlockSpec((DIM, HID), lambda t, hk: (0, 0)),
        ],
        out_specs=pl.BlockSpec((BM, DIM), lambda t, hk: (t, 0)),
        out_shape=jax.ShapeDtypeStruct((N, DIM), jnp.float32),
        scratch_shapes=[pltpu.VMEM((BM, HID), jnp.bfloat16)],
        compiler_params=pltpu.CompilerParams(
            dimension_semantics=("arbitrary", "arbitrary")),
    )(xb16, sw1, sw3, sw2)

    y = _sc_combine(ys, ysh, posf, wf)
    return y.reshape(x.shape).astype(x.dtype)


# speedup vs baseline: 1.6176x; 1.2065x over previous
import functools

import jax
import jax.numpy as jnp
from jax import lax
from jax.experimental import pallas as pl
from jax.experimental.pallas import tpu as pltpu
from jax.experimental.pallas import tpu_sc as plsc

E = 8
DIM = 1024
HID = 2752
N = 2048
BM = 256
NBLK_ROUTED = 24
NBLK_SHARED = N // BM
SHROWS = NBLK_ROUTED * BM
NHK = 2
HB = HID // NHK

NW = 32
TPW = N // NW
CCH = 16


def _gate_body(xf_ref, wg_ref, pos_ref, w_ref, gid_ref, totr_ref):
    xf = xf_ref[...]
    logits = lax.dot_general(xf, wg_ref[...], (((1,), (1,)), ((), ())),
                             preferred_element_type=jnp.float32)
    m = jnp.max(logits, axis=1, keepdims=True)
    ex = jnp.exp(logits - m)
    s = ex / jnp.sum(ex, axis=1, keepdims=True)

    iota = lax.broadcasted_iota(jnp.int32, (N, E), 1)
    m1 = jnp.max(s, axis=1, keepdims=True)
    i1 = jnp.min(jnp.where(s == m1, iota, E), axis=1, keepdims=True)
    top1 = iota == i1
    s2 = jnp.where(top1, -jnp.inf, s)
    m2 = jnp.max(s2, axis=1, keepdims=True)
    i2 = jnp.min(jnp.where(s2 == m2, iota, E), axis=1, keepdims=True)
    h1 = top1.astype(jnp.float32)
    h2 = (iota == i2).astype(jnp.float32)

    nchunk = N // 128
    ci = lax.broadcasted_iota(jnp.int32, (128, 128), 0)
    cj = lax.broadcasted_iota(jnp.int32, (128, 128), 1)
    tril = (ci >= cj).astype(jnp.float32)

    def ranks(h):
        run = jnp.zeros((1, E), jnp.float32)
        outs = []
        for b in range(nchunk):
            hb = h[b * 128:(b + 1) * 128, :]
            incl = lax.dot_general(tril, hb, (((1,), (0,)), ((), ())),
                                   preferred_element_type=jnp.float32)
            outs.append(incl - hb + run)
            run = run + incl[127:128, :]
        return jnp.concatenate(outs, axis=0), run

    r1, tot1 = ranks(h1)
    r2, tot2 = ranks(h2)
    cnt = tot1 + tot2
    nb = jnp.ceil(cnt * (1.0 / BM))
    ei = lax.broadcasted_iota(jnp.int32, (E, E), 0)
    ej = lax.broadcasted_iota(jnp.int32, (E, E), 1)
    sl = (ei < ej).astype(jnp.float32)
    boff = lax.dot_general(nb, sl, (((1,), (0,)), ((), ())),
                           preferred_element_type=jnp.float32)
    cumnext = boff + nb
    totr = cumnext[0:1, E - 1:E]

    base = boff * BM
    pos1 = jnp.sum(h1 * (base + r1), axis=1, keepdims=True)
    pos2 = jnp.sum(h2 * (base + tot1 + r2), axis=1, keepdims=True)
    pos_ref[...] = jnp.concatenate([pos1, pos2], axis=1).astype(jnp.int32)
    w_ref[...] = jnp.concatenate([m1, m2], axis=1)

    bi = lax.broadcasted_iota(jnp.int32, (NBLK_ROUTED, E), 0)
    gid_ref[...] = jnp.sum((bi >= cumnext.astype(jnp.int32)).astype(jnp.int32),
                           axis=1, keepdims=True)
    totr_ref[...] = totr.astype(jnp.int32)


def _swiglu_step(hk, xg, w1_ref, w3_ref, w2t_ref, out_ref):
    h1 = lax.dot_general(xg, w1_ref[...].astype(jnp.bfloat16),
                         (((1,), (1,)), ((), ())),
                         preferred_element_type=jnp.float32)
    h3 = lax.dot_general(xg, w3_ref[...].astype(jnp.bfloat16),
                         (((1,), (1,)), ((), ())),
                         preferred_element_type=jnp.float32)
    hp = (h1 * lax.logistic(h1) * h3).astype(jnp.bfloat16)
    y = lax.dot_general(hp, w2t_ref[...].astype(jnp.bfloat16),
                        (((1,), (0,)), ((), ())),
                        preferred_element_type=jnp.float32)

    @pl.when(hk == 0)
    def _init():
        out_ref[...] = y

    @pl.when(hk != 0)
    def _acc():
        out_ref[...] += y


def _gmm_body(gid_ref, totr_ref, posf_ref, xb_ref, w1_ref, w3_ref,
              w2t_ref, ys_ref, xg_ref):
    b = pl.program_id(0)
    hk = pl.program_id(1)

    @pl.when(b < totr_ref[0])
    def _():
        @pl.when(hk == 0)
        def _gather():
            gp = lax.broadcasted_iota(jnp.int32, (BM, N), 0) + b * BM
            oh = jnp.logical_or(gp == posf_ref[0:1, :],
                                gp == posf_ref[1:2, :]).astype(jnp.bfloat16)
            xg_ref[...] = lax.dot_general(
                oh, xb_ref[...], (((1,), (0,)), ((), ())),
                preferred_element_type=jnp.float32).astype(jnp.bfloat16)

        _swiglu_step(hk, xg_ref[...], w1_ref, w3_ref, w2t_ref, ys_ref)


def _shared_body(xb_ref, w1_ref, w3_ref, w2t_ref, ysh_ref):
    hk = pl.program_id(1)
    _swiglu_step(hk, xb_ref[...], w1_ref, w3_ref, w2t_ref, ysh_ref)


def _sc_combine_body(ys_hbm, ysh_hbm, posf_hbm, wf_hbm, y_hbm,
                     idx0_v, idx1_v, w0_v, w1_v, g0_v, g1_v, sh_v, out_v,
                     sem_g, sem_o):
    wid = lax.axis_index("s") * 2 + lax.axis_index("c")
    base = wid * TPW
    nch = TPW // CCH
    for c in range(nch):
        cb = base + c * CCH
        i0 = pltpu.async_copy(posf_hbm.at[pl.ds(cb, CCH)], idx0_v, sem_g)
        i1 = pltpu.async_copy(posf_hbm.at[pl.ds(N + cb, CCH)], idx1_v, sem_g)
        i2 = pltpu.async_copy(wf_hbm.at[pl.ds(cb, CCH)], w0_v, sem_g)
        i3 = pltpu.async_copy(wf_hbm.at[pl.ds(N + cb, CCH)], w1_v, sem_g)
        i0.wait(); i1.wait(); i2.wait(); i3.wait()
        g0 = pltpu.async_copy(ys_hbm.at[idx0_v], g0_v, sem_g)
        g1 = pltpu.async_copy(ys_hbm.at[idx1_v], g1_v, sem_g)
        gs = pltpu.async_copy(ysh_hbm.at[pl.ds(cb, CCH)], sh_v, sem_g)
        g0.wait(); g1.wait(); gs.wait()
        w0c = w0_v[...]
        w1c = w1_v[...]

        dnums = lax.GatherDimensionNumbers(
            offset_dims=(), collapsed_slice_dims=(0,), start_index_map=(0,))

        def row(r, _):
            ri = jnp.full((CCH, 1), r, jnp.int32)
            w0r = lax.gather(w0c, ri, dnums, (1,),
                             mode=lax.GatherScatterMode.PROMISE_IN_BOUNDS)
            w1r = lax.gather(w1c, ri, dnums, (1,),
                             mode=lax.GatherScatterMode.PROMISE_IN_BOUNDS)
            for v in range(DIM // 16):
                s = pl.ds(v * 16, 16)
                out_v[r, s] = (w0r * g0_v[r, s] + w1r * g1_v[r, s]
                               + sh_v[r, s])
            return 0

        lax.fori_loop(0, CCH, row, 0)
        ow = pltpu.async_copy(out_v, y_hbm.at[pl.ds(cb, CCH)], sem_o)
        ow.wait()


_SC_MESH = plsc.VectorSubcoreMesh(core_axis_name="c", subcore_axis_name="s",
                                  num_cores=2, num_subcores=16)

_sc_combine = functools.partial(
    pl.kernel,
    out_type=jax.ShapeDtypeStruct((N, DIM), jnp.float32),
    mesh=_SC_MESH,
    scratch_types=[
        pltpu.VMEM((CCH,), jnp.int32),
        pltpu.VMEM((CCH,), jnp.int32),
        pltpu.VMEM((CCH,), jnp.float32),
        pltpu.VMEM((CCH,), jnp.float32),
        pltpu.VMEM((CCH, DIM), jnp.float32),
        pltpu.VMEM((CCH, DIM), jnp.float32),
        pltpu.VMEM((CCH, DIM), jnp.float32),
        pltpu.VMEM((CCH, DIM), jnp.float32),
        pltpu.SemaphoreType.DMA,
        pltpu.SemaphoreType.DMA,
    ],
)(_sc_combine_body)


def kernel(x, Wg, W1, W2, W3, sw1, sw2, sw3):
    xf = x.reshape(N, DIM)

    pos2, w2, gid, totr = pl.pallas_call(
        _gate_body,
        out_shape=[
            jax.ShapeDtypeStruct((N, 2), jnp.int32),
            jax.ShapeDtypeStruct((N, 2), jnp.float32),
            jax.ShapeDtypeStruct((NBLK_ROUTED, 1), jnp.int32),
            jax.ShapeDtypeStruct((1, 1), jnp.int32),
        ],
        in_specs=[pl.BlockSpec((N, DIM), lambda: (0, 0)),
                  pl.BlockSpec((E, DIM), lambda: (0, 0))],
        out_specs=[pl.BlockSpec((N, 2), lambda: (0, 0)),
                   pl.BlockSpec((N, 2), lambda: (0, 0)),
                   pl.BlockSpec((NBLK_ROUTED, 1), lambda: (0, 0)),
                   pl.BlockSpec((1, 1), lambda: (0, 0))],
    )(xf, Wg)

    posr = pos2.T.reshape(2, N)
    posf = posr.reshape(2 * N)
    wf = w2.T.reshape(2 * N)
    xb16 = xf.astype(jnp.bfloat16)

    ys = pl.pallas_call(
        _gmm_body,
        grid_spec=pltpu.PrefetchScalarGridSpec(
            num_scalar_prefetch=2,
            grid=(NBLK_ROUTED, NHK),
            in_specs=[
                pl.BlockSpec((2, N), lambda b, hk, gid, totr: (0, 0)),
                pl.BlockSpec((N, DIM), lambda b, hk, gid, totr: (0, 0)),
                pl.BlockSpec((None, HB, DIM),
                             lambda b, hk, gid, totr: (gid[b], hk, 0)),
                pl.BlockSpec((None, HB, DIM),
                             lambda b, hk, gid, totr: (gid[b], hk, 0)),
                pl.BlockSpec((None, HB, DIM),
                             lambda b, hk, gid, totr: (gid[b], hk, 0)),
            ],
            out_specs=pl.BlockSpec((BM, DIM), lambda b, hk, gid, totr: (b, 0)),
            scratch_shapes=[pltpu.VMEM((BM, DIM), jnp.bfloat16)],
        ),
        out_shape=jax.ShapeDtypeStruct((SHROWS, DIM), jnp.float32),
        compiler_params=pltpu.CompilerParams(
            dimension_semantics=("arbitrary", "arbitrary")),
    )(gid.reshape(NBLK_ROUTED), totr.reshape(1), posr, xb16, W1, W3,
      W2.transpose(0, 2, 1))

    ysh = pl.pallas_call(
        _shared_body,
        grid=(N // BM, NHK),
        in_specs=[
            pl.BlockSpec((BM, DIM), lambda t, hk: (t, 0)),
            pl.BlockSpec((HB, DIM), lambda t, hk: (hk, 0)),
            pl.BlockSpec((HB, DIM), lambda t, hk: (hk, 0)),
            pl.BlockSpec((HB, DIM), lambda t, hk: (hk, 0)),
        ],
        out_specs=pl.BlockSpec((BM, DIM), lambda t, hk: (t, 0)),
        out_shape=jax.ShapeDtypeStruct((N, DIM), jnp.float32),
        compiler_params=pltpu.CompilerParams(
            dimension_semantics=("arbitrary", "arbitrary")),
    )(xb16, sw1, sw3, sw2.T)

    y = _sc_combine(ys, ysh, posf, wf)
    return y.reshape(x.shape).astype(x.dtype)
